# baseline (speedup 1.0000x reference)
import functools

import jax
import jax.numpy as jnp
from jax import lax
from jax.experimental import pallas as pl
from jax.experimental.pallas import tpu as pltpu
from jax.experimental.pallas import tpu_sc as plsc

DEPTH = 1000
BATCH = 16384
NC = 2
NS = 16
NW = NC * NS
ROWS_PER_W = BATCH // NW
CHUNK = 32
NCHUNK = ROWS_PER_W // CHUNK
CELEM = CHUNK * DEPTH


def _sc_body(idx_hbm, zeros_hbm, out_hbm, idx_v, buf0, buf1, sem0, sem1):
    wid = lax.axis_index("s") * NC + lax.axis_index("c")
    base = wid * ROWS_PER_W
    pltpu.sync_copy(idx_hbm.at[pl.ds(base, ROWS_PER_W)], idx_v)
    pltpu.sync_copy(zeros_hbm, buf0)
    pltpu.sync_copy(zeros_hbm, buf1)
    bufs = (buf0, buf1)
    sems = (sem0, sem1)
    lane = lax.iota(jnp.int32, 16)
    one_v = jnp.full((16,), 1.0, jnp.float32)
    zero_v = jnp.zeros((16,), jnp.float32)

    def scatter(buf, c, val):
        for g in range(CHUNK // 16):
            pos = (lane + g * 16) * DEPTH + idx_v[pl.ds(c * CHUNK + g * 16, 16)]
            plsc.store_scatter(buf, [pos], val)

    for c in range(NCHUNK):
        b = c % 2
        if c >= 2:
            pltpu.make_async_copy(
                bufs[b], out_hbm.at[pl.ds((base + (c - 2) * CHUNK) * DEPTH, CELEM)],
                sems[b],
            ).wait()
            scatter(bufs[b], c - 2, zero_v)
        scatter(bufs[b], c, one_v)
        pltpu.async_copy(
            bufs[b], out_hbm.at[pl.ds((base + c * CHUNK) * DEPTH, CELEM)], sems[b]
        )
    for b in range(2):
        c = NCHUNK - 2 + b
        pltpu.make_async_copy(
            bufs[b], out_hbm.at[pl.ds((base + c * CHUNK) * DEPTH, CELEM)], sems[b]
        ).wait()


def kernel(X_in, ones):
    del ones
    idx = X_in.astype(jnp.int32)
    zeros = jnp.zeros((CELEM,), jnp.float32)
    mesh = plsc.VectorSubcoreMesh(
        core_axis_name="c", subcore_axis_name="s", num_cores=NC, num_subcores=NS
    )
    run = functools.partial(
        pl.kernel,
        out_type=jax.ShapeDtypeStruct((BATCH * DEPTH,), jnp.float32),
        mesh=mesh,
        compiler_params=pltpu.CompilerParams(needs_layout_passes=False),
        scratch_types=[
            pltpu.VMEM((ROWS_PER_W,), jnp.int32),
            pltpu.VMEM((CELEM,), jnp.float32),
            pltpu.VMEM((CELEM,), jnp.float32),
            pltpu.SemaphoreType.DMA,
            pltpu.SemaphoreType.DMA,
        ],
    )(_sc_body)
    return run(idx, zeros).reshape(BATCH, DEPTH)


# speedup vs baseline: 1.4971x; 1.4971x over previous
import functools

import jax
import jax.numpy as jnp
from jax import lax
from jax.experimental import pallas as pl
from jax.experimental.pallas import tpu as pltpu
from jax.experimental.pallas import tpu_sc as plsc

DEPTH = 1000
BATCH = 16384
NC = 2
NS = 16
NW = NC * NS
ROWS_PER_W = BATCH // NW
CHUNK = 32
NCHUNK = ROWS_PER_W // CHUNK


def _sc_body(idx_hbm, zeros_hbm, out_hbm, idx_v, buf0, buf1, sem0, sem1):
    wid = lax.axis_index("s") * NC + lax.axis_index("c")
    base = wid * ROWS_PER_W
    pltpu.sync_copy(idx_hbm.at[pl.ds(base, ROWS_PER_W)], idx_v)
    pltpu.sync_copy(zeros_hbm, buf0)
    pltpu.sync_copy(zeros_hbm, buf1)
    bufs = (buf0, buf1)
    sems = (sem0, sem1)
    lane = lax.iota(jnp.int32, 16)
    one_v = jnp.full((16,), 1.0, jnp.float32)
    zero_v = jnp.zeros((16,), jnp.float32)

    def scatter(buf, c, val):
        for g in range(CHUNK // 16):
            rows = lane + g * 16
            cols = idx_v[pl.ds(c * CHUNK + g * 16, 16)]
            plsc.store_scatter(buf, [rows, cols], val)

    for c in range(NCHUNK):
        b = c % 2
        if c >= 2:
            pltpu.make_async_copy(
                bufs[b], out_hbm.at[pl.ds(base + (c - 2) * CHUNK, CHUNK)], sems[b]
            ).wait()
            scatter(bufs[b], c - 2, zero_v)
        scatter(bufs[b], c, one_v)
        pltpu.async_copy(
            bufs[b], out_hbm.at[pl.ds(base + c * CHUNK, CHUNK)], sems[b]
        )
    for b in range(2):
        c = NCHUNK - 2 + b
        pltpu.make_async_copy(
            bufs[b], out_hbm.at[pl.ds(base + c * CHUNK, CHUNK)], sems[b]
        ).wait()


def kernel(X_in, ones):
    del ones
    idx = X_in.astype(jnp.int32)
    zeros = jnp.zeros((CHUNK, DEPTH), jnp.float32)
    mesh = plsc.VectorSubcoreMesh(
        core_axis_name="c", subcore_axis_name="s", num_cores=NC, num_subcores=NS
    )
    run = functools.partial(
        pl.kernel,
        out_type=jax.ShapeDtypeStruct((BATCH, DEPTH), jnp.float32),
        mesh=mesh,
        compiler_params=pltpu.CompilerParams(needs_layout_passes=False),
        scratch_types=[
            pltpu.VMEM((ROWS_PER_W,), jnp.int32),
            pltpu.VMEM((CHUNK, DEPTH), jnp.float32),
            pltpu.VMEM((CHUNK, DEPTH), jnp.float32),
            pltpu.SemaphoreType.DMA,
            pltpu.SemaphoreType.DMA,
        ],
    )(_sc_body)
    return run(idx, zeros)


# speedup vs baseline: 1.5006x; 1.0023x over previous
import functools

import jax
import jax.numpy as jnp
from jax import lax
from jax.experimental import pallas as pl
from jax.experimental.pallas import tpu as pltpu
from jax.experimental.pallas import tpu_sc as plsc

DEPTH = 1000
BATCH = 16384
NC = 2
NS = 16
NW = NC * NS
ROWS_PER_W = BATCH // NW
CHUNK = 32
NCHUNK = ROWS_PER_W // CHUNK


def _sc_body(idx_hbm, zeros_hbm, out_hbm, idx_v, buf0, buf1, sem0, sem1):
    wid = lax.axis_index("s") * NC + lax.axis_index("c")
    base = wid * ROWS_PER_W
    pltpu.sync_copy(idx_hbm.at[pl.ds(base, ROWS_PER_W)], idx_v)
    pltpu.sync_copy(zeros_hbm, buf0)
    pltpu.sync_copy(zeros_hbm, buf1)
    bufs = (buf0, buf1)
    sems = (sem0, sem1)
    lane = lax.iota(jnp.int32, 16)
    one_v = jnp.full((16,), 1.0, jnp.float32)
    zero_v = jnp.zeros((16,), jnp.float32)

    def scatter(buf, c, val):
        for g in range(CHUNK // 16):
            rows = lane + g * 16
            cols = idx_v[pl.ds(c * CHUNK + g * 16, 16)]
            plsc.store_scatter(buf, [rows, cols], val)

    for c in range(NCHUNK):
        b = c % 2
        if c >= 2:
            pltpu.make_async_copy(
                bufs[b], out_hbm.at[pl.ds(base + (c - 2) * CHUNK, CHUNK)], sems[b]
            ).wait()
            scatter(bufs[b], c - 2, zero_v)
        scatter(bufs[b], c, one_v)
        pltpu.async_copy(
            bufs[b], out_hbm.at[pl.ds(base + c * CHUNK, CHUNK)], sems[b]
        )
    for b in range(2):
        c = NCHUNK - 2 + b
        pltpu.make_async_copy(
            bufs[b], out_hbm.at[pl.ds(base + c * CHUNK, CHUNK)], sems[b]
        ).wait()


def kernel(X_in, ones):
    del ones
    idx = X_in.astype(jnp.int32)
    zeros = jnp.zeros((CHUNK, DEPTH), jnp.float32)
    mesh = plsc.VectorSubcoreMesh(
        core_axis_name="c", subcore_axis_name="s", num_cores=NC, num_subcores=NS
    )
    run = functools.partial(
        pl.kernel,
        out_type=jax.ShapeDtypeStruct((BATCH, DEPTH), jnp.float32),
        mesh=mesh,
        compiler_params=pltpu.CompilerParams(
            needs_layout_passes=False, use_tc_tiling_on_sc=True
        ),
        scratch_types=[
            pltpu.VMEM((ROWS_PER_W,), jnp.int32),
            pltpu.VMEM((CHUNK, DEPTH), jnp.float32),
            pltpu.VMEM((CHUNK, DEPTH), jnp.float32),
            pltpu.SemaphoreType.DMA,
            pltpu.SemaphoreType.DMA,
        ],
    )(_sc_body)
    return run(idx, zeros)


# speedup vs baseline: 6.3773x; 4.2499x over previous
import jax
import jax.numpy as jnp
from jax.experimental import pallas as pl

DEPTH = 1000
BATCH = 16384
BLOCK = 512


def _onehot_block_t(idx_ref, out_ref):
    idx = idx_ref[0, 0, :]
    iota = jax.lax.broadcasted_iota(jnp.int32, (DEPTH, BLOCK), 0)
    out_ref[...] = (idx[None, :] == iota).astype(jnp.float32)


def kernel(X_in, ones):
    del ones
    grid = BATCH // BLOCK
    idx3 = X_in.astype(jnp.int32).reshape(grid, 1, BLOCK)
    out_t = pl.pallas_call(
        _onehot_block_t,
        grid=(grid,),
        in_specs=[pl.BlockSpec((1, 1, BLOCK), lambda i: (i, 0, 0))],
        out_specs=pl.BlockSpec((DEPTH, BLOCK), lambda i: (0, i)),
        out_shape=jax.ShapeDtypeStruct((DEPTH, BATCH), jnp.float32),
    )(idx3)
    return out_t.T
